# baseline (speedup 1.0000x reference)
import functools
import jax
import jax.numpy as jnp
from jax import lax
from jax.experimental import pallas as pl
from jax.experimental.pallas import tpu as pltpu
from jax.experimental.pallas import tpu_sc as plsc

N = 10000
E = 320000
NODE_DIM = 128
STEPS = 4

E4 = E // 4
ER = E // 128
NC, NS = 2, 16
NW = NC * NS
N_PAD = 10240
C = 128

SCAT_Q, SCAT_R = ER // NS, ER % NS
GATH_Q, GATH_R = ER // NW, ER % NW

_MESH = dict(core_axis_name="c", subcore_axis_name="s",
             num_cores=NC, num_subcores=NS)
_SC_PARAMS = pltpu.CompilerParams(use_tc_tiling_on_sc=False)

f32 = jnp.float32


def _idx_prep_body(ei, s_out, d_out, ds_out):
    s_out[...] = ei[:ER]
    d = ei[ER:]
    d_out[...] = d
    j = lax.broadcasted_iota(jnp.int32, (ER, 128), 1)
    left = jnp.roll(d, -1, axis=1)
    right = jnp.roll(d, 1, axis=1)
    ds_out[...] = jnp.where(j % 2 == 0, left, right)


def _node_pre_body(nf, wn, wl, p_out, r_out):
    x = nf[...]
    p_out[...] = jnp.dot(x, wn[...], preferred_element_type=f32)
    r_out[...] = jnp.dot(x, wl[...], preferred_element_type=f32)


def _q_body(ef, we, q_out):
    q_out[...] = jnp.dot(ef[...], we[...], preferred_element_type=f32)


def _step0_body(g, q, w, e0_out, zs_out):
    e0 = jnp.maximum(g[...] + q[...], 0.0)
    e0_out[...] = e0
    zs_out[...] = jnp.dot(e0, w[...], preferred_element_type=f32)


def _step_mid_body(e0, a, zs, w, zs_out):
    e = jnp.maximum(e0[...] + a[...] - zs[...], 0.0)
    zs_out[...] = jnp.dot(e, w[...], preferred_element_type=f32)


def _step_last_body(e0, a, zs, e_out):
    e_out[...] = jnp.maximum(e0[...] + a[...] - zs[...], 0.0)


def _final_body(r, f0, f1, wf, o_out):
    ff = f0[0] + f1[0]
    o_out[...] = jnp.maximum(r[...] + jnp.dot(ff, wf[...],
                                              preferred_element_type=f32), 0.0)


def _rows_spec(br, w):
    return pl.BlockSpec((br, w), lambda i: (i, 0))


def _full_spec(shape):
    return pl.BlockSpec(shape, lambda i: tuple(0 for _ in shape))


def _tc_idx_prep(ei2):
    return pl.pallas_call(
        _idx_prep_body,
        out_shape=[jax.ShapeDtypeStruct((ER, 128), jnp.int32)] * 3,
    )(ei2)


def _tc_node_pre(n_feat, wn, wl):
    br = 2000
    return pl.pallas_call(
        _node_pre_body,
        grid=(N // br,),
        in_specs=[_rows_spec(br, NODE_DIM), _full_spec((NODE_DIM, 32)),
                  _full_spec((NODE_DIM, 64))],
        out_specs=[_rows_spec(br, 32), _rows_spec(br, 64)],
        out_shape=[jax.ShapeDtypeStruct((N, 32), f32),
                   jax.ShapeDtypeStruct((N, 64), f32)],
    )(n_feat, wn, wl)


def _tc_q(ef, we):
    br = 8000
    return pl.pallas_call(
        _q_body,
        grid=(E // br,),
        in_specs=[_rows_spec(br, 16), _full_spec((16, 32))],
        out_specs=_rows_spec(br, 32),
        out_shape=jax.ShapeDtypeStruct((E, 32), f32),
    )(ef, we)


def _tc_step0(g4, q4, w4p):
    br = 2000
    return pl.pallas_call(
        _step0_body,
        grid=(E4 // br,),
        in_specs=[_rows_spec(br, 128), _rows_spec(br, 128),
                  _full_spec((128, 128))],
        out_specs=[_rows_spec(br, 128), _rows_spec(br, 128)],
        out_shape=[jax.ShapeDtypeStruct((E4, 128), f32),
                   jax.ShapeDtypeStruct((E4, 128), f32)],
    )(g4, q4, w4p)


def _tc_step_mid(e04, a4, zs4, w4p):
    br = 2000
    return pl.pallas_call(
        _step_mid_body,
        grid=(E4 // br,),
        in_specs=[_rows_spec(br, 128)] * 3 + [_full_spec((128, 128))],
        out_specs=_rows_spec(br, 128),
        out_shape=jax.ShapeDtypeStruct((E4, 128), f32),
    )(e04, a4, zs4, w4p)


def _tc_step_last(e04, a4, zs4):
    br = 2000
    return pl.pallas_call(
        _step_last_body,
        grid=(E4 // br,),
        in_specs=[_rows_spec(br, 128)] * 3,
        out_specs=_rows_spec(br, 128),
        out_shape=jax.ShapeDtypeStruct((E4, 128), f32),
    )(e04, a4, zs4)


def _tc_final(r, ffp, wf):
    br = 2000
    return pl.pallas_call(
        _final_body,
        grid=(N // br,),
        in_specs=[_rows_spec(br, 64),
                  pl.BlockSpec((1, br, 32), lambda i: (0, i, 0)),
                  pl.BlockSpec((1, br, 32), lambda i: (1, i, 0)),
                  _full_spec((32, 64))],
        out_specs=_rows_spec(br, 64),
        out_shape=jax.ShapeDtypeStruct((N, 64), f32),
    )(r, ffp, ffp, wf)


def _scatter_pipe(z, idx2, accum, start, n, ib, zb, sl, ss):
    nslot = len(ib)

    def issue(r, s):
        pltpu.async_copy(idx2.at[r], ib[s], sl[s])
        pltpu.async_copy(z.at[pl.ds(r * C, C)], zb[s], sl[s])

    def drain(r, s):
        pltpu.make_async_copy(idx2.at[r], ib[s], sl[s]).wait()
        pltpu.make_async_copy(z.at[pl.ds(r * C, C)], zb[s], sl[s]).wait()

    def wait_scat(s):
        pltpu.make_async_copy(zb[s], accum.at[ib[s]], ss[s]).wait()

    issue(start, 0)

    def step(j, s):
        nxt = (s + 1) % nslot
        @pl.when(j + 1 >= nslot)
        def _():
            wait_scat(nxt)
        @pl.when(j + 1 < n)
        def _():
            issue(start + j + 1, nxt)
        drain(start + j, s)
        pltpu.async_copy(zb[s], accum.at[ib[s]], ss[s], add=True)

    def body(j, _):
        for s in range(nslot):
            @pl.when(j % nslot == s)
            def _(s=s):
                step(j, s)
        return 0

    lax.fori_loop(0, n, body, 0)
    for k in range(1, nslot):
        if n - k >= 0:
            wait_scat((n - k) % nslot)


def _scatter_tail(z, idx2, accum, worker, ib, zb):
    @pl.when(worker < GATH_R)
    def _():
        r = NW * GATH_Q + worker
        pltpu.sync_copy(idx2.at[r], ib)
        pltpu.sync_copy(z.at[pl.ds(r * C, C)], zb)
        pltpu.sync_copy(zb, accum.at[ib], add=True)


def _gather_pipe(tbl, idx2, out, start, n, ib, gb, sl, sg):
    pltpu.async_copy(idx2.at[start], ib[0], sl[0])

    def step(j, s):
        @pl.when(j > 0)
        def _():
            pltpu.make_async_copy(tbl.at[ib[1 - s]], gb[1 - s],
                                  sg[1 - s]).wait()
            pltpu.sync_copy(gb[1 - s], out.at[pl.ds((start + j - 1) * C, C)])
        pltpu.make_async_copy(idx2.at[start + j], ib[s], sl[s]).wait()
        pltpu.async_copy(tbl.at[ib[s]], gb[s], sg[s])
        @pl.when(j + 1 < n)
        def _():
            pltpu.async_copy(idx2.at[start + j + 1], ib[1 - s], sl[1 - s])

    def body(j, _):
        @pl.when(j % 2 == 0)
        def _():
            step(j, 0)
        @pl.when(j % 2 == 1)
        def _():
            step(j, 1)
        return 0

    lax.fori_loop(0, n, body, 0)
    s_last = (n - 1) % 2
    for s in (0, 1):
        @pl.when(s_last == s)
        def _(s=s):
            pltpu.make_async_copy(tbl.at[ib[s]], gb[s], sg[s]).wait()
            pltpu.sync_copy(gb[s], out.at[pl.ds((start + n - 1) * C, C)])


def _gather_tail(tbl, idx2, out, worker, ib, gb, sem):
    @pl.when(worker < GATH_R)
    def _():
        r = NW * GATH_Q + worker
        pltpu.sync_copy(idx2.at[r], ib)
        pltpu.async_copy(tbl.at[ib], gb, sem).wait()
        pltpu.sync_copy(gb, out.at[pl.ds(r * C, C)])


_SC_SCRATCH = (
    [pltpu.VMEM((C,), jnp.int32)] * 4 + [pltpu.VMEM((C, 32), f32)] * 4
    + [pltpu.SemaphoreType.DMA] * 8
)


def _sc_gather(p_tbl, src2):
    @functools.partial(
        pl.kernel,
        out_type=jax.ShapeDtypeStruct((E, 32), f32),
        mesh=plsc.VectorSubcoreMesh(**_MESH),
        compiler_params=_SC_PARAMS,
        scratch_types=_SC_SCRATCH,
    )
    def k(tbl, idx2, out, ib0, ib1, ib2, ib3, gb0, gb1, gb2, gb3,
          sl0, sl1, sl2, sl3, sg0, sg1, sg2, sg3):
        wid = lax.axis_index("s") * NC + lax.axis_index("c")
        _gather_pipe(tbl, idx2, out, wid * GATH_Q, GATH_Q,
                     (ib0, ib1), (gb0, gb1), (sl0, sl1), (sg0, sg1))
        _gather_tail(tbl, idx2, out, wid, ib0, gb0, sg0)

    return k(p_tbl, src2)


def _sc_step(zs, dstr2, src2, zrows):
    @functools.partial(
        pl.kernel,
        out_type=jax.ShapeDtypeStruct((E, 32), f32),
        mesh=plsc.VectorSubcoreMesh(**_MESH),
        compiler_params=_SC_PARAMS,
        scratch_types=[pltpu.VMEM_SHARED((N_PAD, 32), f32)] + _SC_SCRATCH,
    )
    def k(z, dstr_i, src_i, zr, a, accum,
          ib0, ib1, ib2, ib3, zb0, zb1, zb2, zb3,
          sl0, sl1, sl2, sl3, sg0, sg1, sg2, sg3):
        cid = lax.axis_index("c")
        tid = lax.axis_index("s")
        pltpu.sync_copy(zr, accum.at[pl.ds(tid * 640, 640)])
        plsc.subcore_barrier()
        _scatter_pipe(z, dstr_i, accum, tid * SCAT_Q, SCAT_Q,
                      (ib0, ib1, ib2, ib3), (zb0, zb1, zb2, zb3),
                      (sl0, sl1, sl2, sl3), (sg0, sg1, sg2, sg3))
        @pl.when(tid < SCAT_R)
        def _():
            r = NS * SCAT_Q + tid
            pltpu.sync_copy(dstr_i.at[r], ib0)
            pltpu.sync_copy(z.at[pl.ds(r * C, C)], zb0)
            pltpu.sync_copy(zb0, accum.at[ib0], add=True)
        plsc.subcore_barrier()
        wid = tid * NC + cid
        _gather_pipe(accum, src_i, a, wid * GATH_Q, GATH_Q,
                     (ib0, ib1), (zb0, zb1), (sl0, sl1), (sg0, sg1))
        _gather_tail(accum, src_i, a, wid, ib0, zb0, sg0)

    return k(zs, dstr2, src2, zrows)


def _sc_final(e4f, dst2, zrows):
    @functools.partial(
        pl.kernel,
        out_type=jax.ShapeDtypeStruct((NC, N_PAD, 32), f32),
        mesh=plsc.VectorSubcoreMesh(**_MESH),
        compiler_params=_SC_PARAMS,
        scratch_types=[pltpu.VMEM_SHARED((N_PAD, 32), f32)] + _SC_SCRATCH,
    )
    def k(e, dst_i, zr, ffp, accum,
          ib0, ib1, ib2, ib3, zb0, zb1, zb2, zb3,
          sl0, sl1, sl2, sl3, sg0, sg1, sg2, sg3):
        cid = lax.axis_index("c")
        tid = lax.axis_index("s")
        pltpu.sync_copy(zr, accum.at[pl.ds(tid * 640, 640)])
        plsc.subcore_barrier()
        wid = tid * NC + cid
        _scatter_pipe(e, dst_i, accum, wid * GATH_Q, GATH_Q,
                      (ib0, ib1, ib2, ib3), (zb0, zb1, zb2, zb3),
                      (sl0, sl1, sl2, sl3), (sg0, sg1, sg2, sg3))
        _scatter_tail(e, dst_i, accum, wid, ib0, zb0)
        plsc.subcore_barrier()
        pltpu.sync_copy(accum.at[pl.ds(tid * 640, 640)],
                        ffp.at[cid, pl.ds(tid * 640, 640)])

    return k(e4f, dst2, zrows)


def kernel(n_feat, e_feat, edge_index, W_init, W_eupd, W_last):
    ei2 = edge_index.reshape(2 * ER, 128)
    src2, dst2, dstr2 = _tc_idx_prep(ei2)

    w4p = jnp.kron(jnp.eye(4, dtype=f32), W_eupd)[jnp.arange(128) ^ 32]
    zrows = jnp.zeros((640, 32), f32)

    p_tbl, r = _tc_node_pre(n_feat, W_init[:NODE_DIM], W_last[:NODE_DIM])
    q4 = _tc_q(e_feat, W_init[NODE_DIM:]).reshape(E4, 128)
    g4 = _sc_gather(p_tbl, src2).reshape(E4, 128)
    e04, zs4 = _tc_step0(g4, q4, w4p)
    for _ in range(STEPS - 1):
        a4 = _sc_step(zs4.reshape(E, 32), dstr2, src2, zrows).reshape(E4, 128)
        zs4 = _tc_step_mid(e04, a4, zs4, w4p)
    a4 = _sc_step(zs4.reshape(E, 32), dstr2, src2, zrows).reshape(E4, 128)
    e4f = _tc_step_last(e04, a4, zs4)
    ffp = _sc_final(e4f.reshape(E, 32), dst2, zrows)
    out = _tc_final(r, ffp, W_last[NODE_DIM:])
    return out


# speedup vs baseline: 1.1364x; 1.1364x over previous
import functools
import jax
import jax.numpy as jnp
from jax import lax
from jax.experimental import pallas as pl
from jax.experimental.pallas import tpu as pltpu
from jax.experimental.pallas import tpu_sc as plsc

N = 10000
E = 320000
NODE_DIM = 128
STEPS = 4

E4 = E // 4
ER = E // 128
NC, NS = 2, 16
NW = NC * NS
N_PAD = 10240
C = 128

SCAT_Q, SCAT_R = ER // NS, ER % NS
GATH_Q, GATH_R = ER // NW, ER % NW

_MESH = dict(core_axis_name="c", subcore_axis_name="s",
             num_cores=NC, num_subcores=NS)
_SC_PARAMS = pltpu.CompilerParams(use_tc_tiling_on_sc=False)

f32 = jnp.float32


def _idx_prep_body(ei, s_out, d_out, ds_out):
    s_out[...] = ei[:ER]
    d = ei[ER:]
    d_out[...] = d
    j = lax.broadcasted_iota(jnp.int32, (ER, 128), 1)
    left = jnp.roll(d, -1, axis=1)
    right = jnp.roll(d, 1, axis=1)
    ds_out[...] = jnp.where(j % 2 == 0, left, right)


def _node_pre_body(nf, wn, wl, p_out, r_out):
    x = nf[...]
    p_out[...] = jnp.dot(x, wn[...], preferred_element_type=f32)
    r_out[...] = jnp.dot(x, wl[...], preferred_element_type=f32)


def _q_body(ef, we, q_out):
    q_out[...] = jnp.dot(ef[...], we[...], preferred_element_type=f32)


def _step0_body(g, q, w, e0_out, zs_out):
    e0 = jnp.maximum(g[...] + q[...], 0.0)
    e0_out[...] = e0
    zs_out[...] = jnp.dot(e0, w[...], preferred_element_type=f32)


def _step_mid_body(e0, a, zs, w, zs_out):
    e = jnp.maximum(e0[...] + a[...] - zs[...], 0.0)
    zs_out[...] = jnp.dot(e, w[...], preferred_element_type=f32)


def _step_last_body(e0, a, zs, e_out):
    e_out[...] = jnp.maximum(e0[...] + a[...] - zs[...], 0.0)


def _final_body(r, f0, f1, wf, o_out):
    ff = f0[0] + f1[0]
    o_out[...] = jnp.maximum(r[...] + jnp.dot(ff, wf[...],
                                              preferred_element_type=f32), 0.0)


def _rows_spec(br, w):
    return pl.BlockSpec((br, w), lambda i: (i, 0))


def _full_spec(shape):
    return pl.BlockSpec(shape, lambda i: tuple(0 for _ in shape))


def _tc_idx_prep(ei2):
    return pl.pallas_call(
        _idx_prep_body,
        out_shape=[jax.ShapeDtypeStruct((ER, 128), jnp.int32)] * 3,
    )(ei2)


def _tc_node_pre(n_feat, wn, wl):
    br = 2000
    return pl.pallas_call(
        _node_pre_body,
        grid=(N // br,),
        in_specs=[_rows_spec(br, NODE_DIM), _full_spec((NODE_DIM, 32)),
                  _full_spec((NODE_DIM, 64))],
        out_specs=[_rows_spec(br, 32), _rows_spec(br, 64)],
        out_shape=[jax.ShapeDtypeStruct((N, 32), f32),
                   jax.ShapeDtypeStruct((N, 64), f32)],
    )(n_feat, wn, wl)


def _tc_q(ef4, we4):
    br = 2000
    return pl.pallas_call(
        _q_body,
        grid=(E4 // br,),
        in_specs=[_rows_spec(br, 64), _full_spec((64, 128))],
        out_specs=_rows_spec(br, 128),
        out_shape=jax.ShapeDtypeStruct((E4, 128), f32),
    )(ef4, we4)


def _tc_step0(g4, q4, w4p):
    br = 2000
    return pl.pallas_call(
        _step0_body,
        grid=(E4 // br,),
        in_specs=[_rows_spec(br, 128), _rows_spec(br, 128),
                  _full_spec((128, 128))],
        out_specs=[_rows_spec(br, 128), _rows_spec(br, 128)],
        out_shape=[jax.ShapeDtypeStruct((E4, 128), f32),
                   jax.ShapeDtypeStruct((E4, 128), f32)],
    )(g4, q4, w4p)


def _tc_step_mid(e04, a4, zs4, w4p):
    br = 2000
    return pl.pallas_call(
        _step_mid_body,
        grid=(E4 // br,),
        in_specs=[_rows_spec(br, 128)] * 3 + [_full_spec((128, 128))],
        out_specs=_rows_spec(br, 128),
        out_shape=jax.ShapeDtypeStruct((E4, 128), f32),
    )(e04, a4, zs4, w4p)


def _tc_step_last(e04, a4, zs4):
    br = 2000
    return pl.pallas_call(
        _step_last_body,
        grid=(E4 // br,),
        in_specs=[_rows_spec(br, 128)] * 3,
        out_specs=_rows_spec(br, 128),
        out_shape=jax.ShapeDtypeStruct((E4, 128), f32),
    )(e04, a4, zs4)


def _tc_final(r, ffp, wf):
    br = 2000
    return pl.pallas_call(
        _final_body,
        grid=(N // br,),
        in_specs=[_rows_spec(br, 64),
                  pl.BlockSpec((1, br, 32), lambda i: (0, i, 0)),
                  pl.BlockSpec((1, br, 32), lambda i: (1, i, 0)),
                  _full_spec((32, 64))],
        out_specs=_rows_spec(br, 64),
        out_shape=jax.ShapeDtypeStruct((N, 64), f32),
    )(r, ffp, ffp, wf)


def _scatter_pipe(z, idx2, accum, start, n, ib, zb, sl, ss):
    nslot = len(ib)

    def issue(r, s):
        pltpu.async_copy(idx2.at[r], ib[s], sl[s])
        pltpu.async_copy(z.at[pl.ds(r * C, C)], zb[s], sl[s])

    def drain(r, s):
        pltpu.make_async_copy(idx2.at[r], ib[s], sl[s]).wait()
        pltpu.make_async_copy(z.at[pl.ds(r * C, C)], zb[s], sl[s]).wait()

    def wait_scat(s):
        pltpu.make_async_copy(zb[s], accum.at[ib[s]], ss[s]).wait()

    issue(start, 0)

    def step(j, s):
        nxt = (s + 1) % nslot
        @pl.when(j + 1 >= nslot)
        def _():
            wait_scat(nxt)
        @pl.when(j + 1 < n)
        def _():
            issue(start + j + 1, nxt)
        drain(start + j, s)
        pltpu.async_copy(zb[s], accum.at[ib[s]], ss[s], add=True)

    def body(j, _):
        for s in range(nslot):
            @pl.when(j % nslot == s)
            def _(s=s):
                step(j, s)
        return 0

    lax.fori_loop(0, n, body, 0)
    for k in range(1, nslot):
        if n - k >= 0:
            wait_scat((n - k) % nslot)


def _scatter_tail(z, idx2, accum, worker, ib, zb):
    @pl.when(worker < GATH_R)
    def _():
        r = NW * GATH_Q + worker
        pltpu.sync_copy(idx2.at[r], ib)
        pltpu.sync_copy(z.at[pl.ds(r * C, C)], zb)
        pltpu.sync_copy(zb, accum.at[ib], add=True)


def _gather_pipe(tbl, idx2, out, start, n, ib, gb, sl, sg):
    pltpu.async_copy(idx2.at[start], ib[0], sl[0])

    def step(j, s):
        @pl.when(j > 0)
        def _():
            pltpu.make_async_copy(tbl.at[ib[1 - s]], gb[1 - s],
                                  sg[1 - s]).wait()
            pltpu.sync_copy(gb[1 - s], out.at[pl.ds((start + j - 1) * C, C)])
        pltpu.make_async_copy(idx2.at[start + j], ib[s], sl[s]).wait()
        pltpu.async_copy(tbl.at[ib[s]], gb[s], sg[s])
        @pl.when(j + 1 < n)
        def _():
            pltpu.async_copy(idx2.at[start + j + 1], ib[1 - s], sl[1 - s])

    def body(j, _):
        @pl.when(j % 2 == 0)
        def _():
            step(j, 0)
        @pl.when(j % 2 == 1)
        def _():
            step(j, 1)
        return 0

    lax.fori_loop(0, n, body, 0)
    s_last = (n - 1) % 2
    for s in (0, 1):
        @pl.when(s_last == s)
        def _(s=s):
            pltpu.make_async_copy(tbl.at[ib[s]], gb[s], sg[s]).wait()
            pltpu.sync_copy(gb[s], out.at[pl.ds((start + n - 1) * C, C)])


def _gather_tail(tbl, idx2, out, worker, ib, gb, sem):
    @pl.when(worker < GATH_R)
    def _():
        r = NW * GATH_Q + worker
        pltpu.sync_copy(idx2.at[r], ib)
        pltpu.async_copy(tbl.at[ib], gb, sem).wait()
        pltpu.sync_copy(gb, out.at[pl.ds(r * C, C)])


_SC_SCRATCH = (
    [pltpu.VMEM((C,), jnp.int32)] * 4 + [pltpu.VMEM((C, 32), f32)] * 4
    + [pltpu.SemaphoreType.DMA] * 8
)


def _sc_gather(p_tbl, src2):
    @functools.partial(
        pl.kernel,
        out_type=jax.ShapeDtypeStruct((E, 32), f32),
        mesh=plsc.VectorSubcoreMesh(**_MESH),
        compiler_params=_SC_PARAMS,
        scratch_types=_SC_SCRATCH,
    )
    def k(tbl, idx2, out, ib0, ib1, ib2, ib3, gb0, gb1, gb2, gb3,
          sl0, sl1, sl2, sl3, sg0, sg1, sg2, sg3):
        wid = lax.axis_index("s") * NC + lax.axis_index("c")
        _gather_pipe(tbl, idx2, out, wid * GATH_Q, GATH_Q,
                     (ib0, ib1), (gb0, gb1), (sl0, sl1), (sg0, sg1))
        _gather_tail(tbl, idx2, out, wid, ib0, gb0, sg0)

    return k(p_tbl, src2)


def _sc_step(zs, dstr2, src2, zrows):
    @functools.partial(
        pl.kernel,
        out_type=jax.ShapeDtypeStruct((E, 32), f32),
        mesh=plsc.VectorSubcoreMesh(**_MESH),
        compiler_params=_SC_PARAMS,
        scratch_types=[pltpu.VMEM_SHARED((N_PAD, 32), f32)] + _SC_SCRATCH,
    )
    def k(z, dstr_i, src_i, zr, a, accum,
          ib0, ib1, ib2, ib3, zb0, zb1, zb2, zb3,
          sl0, sl1, sl2, sl3, sg0, sg1, sg2, sg3):
        cid = lax.axis_index("c")
        tid = lax.axis_index("s")
        pltpu.sync_copy(zr, accum.at[pl.ds(tid * 640, 640)])
        plsc.subcore_barrier()
        _scatter_pipe(z, dstr_i, accum, tid * SCAT_Q, SCAT_Q,
                      (ib0, ib1, ib2, ib3), (zb0, zb1, zb2, zb3),
                      (sl0, sl1, sl2, sl3), (sg0, sg1, sg2, sg3))
        @pl.when(tid < SCAT_R)
        def _():
            r = NS * SCAT_Q + tid
            pltpu.sync_copy(dstr_i.at[r], ib0)
            pltpu.sync_copy(z.at[pl.ds(r * C, C)], zb0)
            pltpu.sync_copy(zb0, accum.at[ib0], add=True)
        plsc.subcore_barrier()
        wid = tid * NC + cid
        _gather_pipe(accum, src_i, a, wid * GATH_Q, GATH_Q,
                     (ib0, ib1), (zb0, zb1), (sl0, sl1), (sg0, sg1))
        _gather_tail(accum, src_i, a, wid, ib0, zb0, sg0)

    return k(zs, dstr2, src2, zrows)


def _sc_final(e4f, dst2, zrows):
    @functools.partial(
        pl.kernel,
        out_type=jax.ShapeDtypeStruct((NC, N_PAD, 32), f32),
        mesh=plsc.VectorSubcoreMesh(**_MESH),
        compiler_params=_SC_PARAMS,
        scratch_types=[pltpu.VMEM_SHARED((N_PAD, 32), f32)] + _SC_SCRATCH,
    )
    def k(e, dst_i, zr, ffp, accum,
          ib0, ib1, ib2, ib3, zb0, zb1, zb2, zb3,
          sl0, sl1, sl2, sl3, sg0, sg1, sg2, sg3):
        cid = lax.axis_index("c")
        tid = lax.axis_index("s")
        pltpu.sync_copy(zr, accum.at[pl.ds(tid * 640, 640)])
        plsc.subcore_barrier()
        wid = tid * NC + cid
        _scatter_pipe(e, dst_i, accum, wid * GATH_Q, GATH_Q,
                      (ib0, ib1, ib2, ib3), (zb0, zb1, zb2, zb3),
                      (sl0, sl1, sl2, sl3), (sg0, sg1, sg2, sg3))
        _scatter_tail(e, dst_i, accum, wid, ib0, zb0)
        plsc.subcore_barrier()
        pltpu.sync_copy(accum.at[pl.ds(tid * 640, 640)],
                        ffp.at[cid, pl.ds(tid * 640, 640)])

    return k(e4f, dst2, zrows)


def kernel(n_feat, e_feat, edge_index, W_init, W_eupd, W_last):
    ei2 = edge_index.reshape(2 * ER, 128)
    src2, dst2, dstr2 = _tc_idx_prep(ei2)

    w4p = jnp.kron(jnp.eye(4, dtype=f32), W_eupd)[jnp.arange(128) ^ 32]
    we4 = jnp.kron(jnp.eye(4, dtype=f32), W_init[NODE_DIM:])
    zrows = jnp.zeros((640, 32), f32)

    p_tbl, r = _tc_node_pre(n_feat, W_init[:NODE_DIM], W_last[:NODE_DIM])
    q4 = _tc_q(e_feat.reshape(E4, 64), we4)
    g4 = _sc_gather(p_tbl, src2).reshape(E4, 128)
    e04, zs4 = _tc_step0(g4, q4, w4p)
    for _ in range(STEPS - 1):
        a4 = _sc_step(zs4.reshape(E, 32), dstr2, src2, zrows).reshape(E4, 128)
        zs4 = _tc_step_mid(e04, a4, zs4, w4p)
    a4 = _sc_step(zs4.reshape(E, 32), dstr2, src2, zrows).reshape(E4, 128)
    e4f = _tc_step_last(e04, a4, zs4)
    ffp = _sc_final(e4f.reshape(E, 32), dst2, zrows)
    out = _tc_final(r, ffp, W_last[NODE_DIM:])
    return out


# speedup vs baseline: 1.1702x; 1.0298x over previous
import functools
import jax
import jax.numpy as jnp
from jax import lax
from jax.experimental import pallas as pl
from jax.experimental.pallas import tpu as pltpu
from jax.experimental.pallas import tpu_sc as plsc

N = 10000
E = 320000
NODE_DIM = 128
STEPS = 4

E4 = E // 4
ER = E // 128
NC, NS = 2, 16
NW = NC * NS
N_PAD = 10240
C = 128

SCAT_Q, SCAT_R = ER // NS, ER % NS
GATH_Q, GATH_R = ER // NW, ER % NW

_MESH = dict(core_axis_name="c", subcore_axis_name="s",
             num_cores=NC, num_subcores=NS)
_SC_PARAMS = pltpu.CompilerParams(use_tc_tiling_on_sc=False)

f32 = jnp.float32


def _idx_prep_body(ei, s_out, d_out, ds_out):
    s_out[...] = ei[:ER]
    d = ei[ER:]
    d_out[...] = d
    j = lax.broadcasted_iota(jnp.int32, (ER, 128), 1)
    left = jnp.roll(d, -1, axis=1)
    right = jnp.roll(d, 1, axis=1)
    ds_out[...] = jnp.where(j % 2 == 0, left, right)


def _node_pre_body(nf, wn, wl, p_out, r_out):
    x = nf[...]
    p_out[...] = jnp.dot(x, wn[...], preferred_element_type=f32)
    r_out[...] = jnp.dot(x, wl[...], preferred_element_type=f32)


def _q_body(ef, we, q_out):
    q_out[...] = jnp.dot(ef[...], we[...], preferred_element_type=f32)


def _step0_body(g, q, w, e0_out, zs_out):
    e0 = jnp.maximum(g[...] + q[...], 0.0)
    e0_out[...] = e0
    zs_out[...] = jnp.dot(e0, w[...], preferred_element_type=f32)


def _step_mid_body(e0, a, zs, w, zs_out):
    e = jnp.maximum(e0[...] + a[...] - zs[...], 0.0)
    zs_out[...] = jnp.dot(e, w[...], preferred_element_type=f32)


def _step_last_body(e0, a, zs, e_out):
    e_out[...] = jnp.maximum(e0[...] + a[...] - zs[...], 0.0)


def _final_body(r, f0, f1, wf, o_out):
    ff = f0[0] + f1[0]
    o_out[...] = jnp.maximum(r[...] + jnp.dot(ff, wf[...],
                                              preferred_element_type=f32), 0.0)


def _rows_spec(br, w):
    return pl.BlockSpec((br, w), lambda i: (i, 0))


def _full_spec(shape):
    return pl.BlockSpec(shape, lambda i: tuple(0 for _ in shape))


def _tc_idx_prep(ei2):
    return pl.pallas_call(
        _idx_prep_body,
        out_shape=[jax.ShapeDtypeStruct((ER, 128), jnp.int32)] * 3,
    )(ei2)


def _tc_node_pre(n_feat, wn, wl):
    br = 2000
    return pl.pallas_call(
        _node_pre_body,
        grid=(N // br,),
        in_specs=[_rows_spec(br, NODE_DIM), _full_spec((NODE_DIM, 32)),
                  _full_spec((NODE_DIM, 64))],
        out_specs=[_rows_spec(br, 32), _rows_spec(br, 64)],
        out_shape=[jax.ShapeDtypeStruct((N, 32), f32),
                   jax.ShapeDtypeStruct((N, 64), f32)],
    )(n_feat, wn, wl)


def _tc_q(ef4, we4):
    br = 2000
    return pl.pallas_call(
        _q_body,
        grid=(E4 // br,),
        in_specs=[_rows_spec(br, 64), _full_spec((64, 128))],
        out_specs=_rows_spec(br, 128),
        out_shape=jax.ShapeDtypeStruct((E4, 128), f32),
    )(ef4, we4)


def _tc_step0(g4, q4, w4p):
    br = 2000
    return pl.pallas_call(
        _step0_body,
        grid=(E4 // br,),
        in_specs=[_rows_spec(br, 128), _rows_spec(br, 128),
                  _full_spec((128, 128))],
        out_specs=[_rows_spec(br, 128), _rows_spec(br, 128)],
        out_shape=[jax.ShapeDtypeStruct((E4, 128), f32),
                   jax.ShapeDtypeStruct((E4, 128), f32)],
    )(g4, q4, w4p)


def _tc_step_mid(e04, a4, zs4, w4p):
    br = 2000
    return pl.pallas_call(
        _step_mid_body,
        grid=(E4 // br,),
        in_specs=[_rows_spec(br, 128)] * 3 + [_full_spec((128, 128))],
        out_specs=_rows_spec(br, 128),
        out_shape=jax.ShapeDtypeStruct((E4, 128), f32),
    )(e04, a4, zs4, w4p)


def _tc_step_last(e04, a4, zs4):
    br = 2000
    return pl.pallas_call(
        _step_last_body,
        grid=(E4 // br,),
        in_specs=[_rows_spec(br, 128)] * 3,
        out_specs=_rows_spec(br, 128),
        out_shape=jax.ShapeDtypeStruct((E4, 128), f32),
    )(e04, a4, zs4)


def _tc_final(r, ffp, wf):
    br = 2000
    return pl.pallas_call(
        _final_body,
        grid=(N // br,),
        in_specs=[_rows_spec(br, 64),
                  pl.BlockSpec((1, br, 32), lambda i: (0, i, 0)),
                  pl.BlockSpec((1, br, 32), lambda i: (1, i, 0)),
                  _full_spec((32, 64))],
        out_specs=_rows_spec(br, 64),
        out_shape=jax.ShapeDtypeStruct((N, 64), f32),
    )(r, ffp, ffp, wf)


def _scatter_pipe(z, idx2, accum, start, n, ib, zb, sl, ss):
    nslot = len(ib)

    def issue(r, s):
        pltpu.async_copy(idx2.at[r], ib[s], sl[s])
        pltpu.async_copy(z.at[pl.ds(r * C, C)], zb[s], sl[s])

    def drain(r, s):
        pltpu.make_async_copy(idx2.at[r], ib[s], sl[s]).wait()
        pltpu.make_async_copy(z.at[pl.ds(r * C, C)], zb[s], sl[s]).wait()

    def wait_scat(s):
        pltpu.make_async_copy(zb[s], accum.at[ib[s]], ss[s]).wait()

    issue(start, 0)

    def step(j, s):
        nxt = (s + 1) % nslot
        @pl.when(j + 1 >= nslot)
        def _():
            wait_scat(nxt)
        @pl.when(j + 1 < n)
        def _():
            issue(start + j + 1, nxt)
        drain(start + j, s)
        pltpu.async_copy(zb[s], accum.at[ib[s]], ss[s], add=True)

    def body(j, _):
        for s in range(nslot):
            @pl.when(j % nslot == s)
            def _(s=s):
                step(j, s)
        return 0

    lax.fori_loop(0, n, body, 0)
    for k in range(1, nslot):
        if n - k >= 0:
            wait_scat((n - k) % nslot)


def _scatter_tail(z, idx2, accum, worker, ib, zb):
    @pl.when(worker < GATH_R)
    def _():
        r = NW * GATH_Q + worker
        pltpu.sync_copy(idx2.at[r], ib)
        pltpu.sync_copy(z.at[pl.ds(r * C, C)], zb)
        pltpu.sync_copy(zb, accum.at[ib], add=True)


def _gather_pipe(tbl, idx2, out, start, n, ib, gb, sl, sg, sw):
    nslot = len(ib)

    def wait_write(r, s):
        pltpu.make_async_copy(gb[s], out.at[pl.ds(r * C, C)], sw[s]).wait()

    pltpu.async_copy(idx2.at[start], ib[0], sl[0])

    def step(j, s):
        nxt = (s + 1) % nslot
        @pl.when(j + 1 >= nslot)
        def _():
            wait_write(start + j + 1 - nslot, nxt)
        @pl.when(j + 1 < n)
        def _():
            pltpu.async_copy(idx2.at[start + j + 1], ib[nxt], sl[nxt])
        pltpu.make_async_copy(idx2.at[start + j], ib[s], sl[s]).wait()
        pltpu.async_copy(tbl.at[ib[s]], gb[s], sg[s])
        pltpu.make_async_copy(tbl.at[ib[s]], gb[s], sg[s]).wait()
        pltpu.async_copy(gb[s], out.at[pl.ds((start + j) * C, C)], sw[s])

    def body(j, _):
        for s in range(nslot):
            @pl.when(j % nslot == s)
            def _(s=s):
                step(j, s)
        return 0

    lax.fori_loop(0, n, body, 0)
    for k in range(1, nslot):
        if n - k >= 0:
            wait_write(start + n - k, (n - k) % nslot)


def _gather_tail(tbl, idx2, out, worker, ib, gb, sem):
    @pl.when(worker < GATH_R)
    def _():
        r = NW * GATH_Q + worker
        pltpu.sync_copy(idx2.at[r], ib)
        pltpu.async_copy(tbl.at[ib], gb, sem).wait()
        pltpu.sync_copy(gb, out.at[pl.ds(r * C, C)])


_SC_SCRATCH = (
    [pltpu.VMEM((C,), jnp.int32)] * 4 + [pltpu.VMEM((C, 32), f32)] * 4
    + [pltpu.SemaphoreType.DMA] * 12
)


def _sc_gather(p_tbl, src2):
    @functools.partial(
        pl.kernel,
        out_type=jax.ShapeDtypeStruct((E, 32), f32),
        mesh=plsc.VectorSubcoreMesh(**_MESH),
        compiler_params=_SC_PARAMS,
        scratch_types=_SC_SCRATCH,
    )
    def k(tbl, idx2, out, ib0, ib1, ib2, ib3, gb0, gb1, gb2, gb3,
          sl0, sl1, sl2, sl3, sg0, sg1, sg2, sg3, sw0, sw1, sw2, sw3):
        wid = lax.axis_index("s") * NC + lax.axis_index("c")
        _gather_pipe(tbl, idx2, out, wid * GATH_Q, GATH_Q,
                     (ib0, ib1, ib2, ib3), (gb0, gb1, gb2, gb3),
                     (sl0, sl1, sl2, sl3), (sg0, sg1, sg2, sg3),
                     (sw0, sw1, sw2, sw3))
        _gather_tail(tbl, idx2, out, wid, ib0, gb0, sg0)

    return k(p_tbl, src2)


def _sc_step(zs, dstr2, src2, zrows):
    @functools.partial(
        pl.kernel,
        out_type=jax.ShapeDtypeStruct((E, 32), f32),
        mesh=plsc.VectorSubcoreMesh(**_MESH),
        compiler_params=_SC_PARAMS,
        scratch_types=[pltpu.VMEM_SHARED((N_PAD, 32), f32)] + _SC_SCRATCH,
    )
    def k(z, dstr_i, src_i, zr, a, accum,
          ib0, ib1, ib2, ib3, zb0, zb1, zb2, zb3,
          sl0, sl1, sl2, sl3, sg0, sg1, sg2, sg3, sw0, sw1, sw2, sw3):
        cid = lax.axis_index("c")
        tid = lax.axis_index("s")
        pltpu.sync_copy(zr, accum.at[pl.ds(tid * 640, 640)])
        plsc.subcore_barrier()
        _scatter_pipe(z, dstr_i, accum, tid * SCAT_Q, SCAT_Q,
                      (ib0, ib1, ib2, ib3), (zb0, zb1, zb2, zb3),
                      (sl0, sl1, sl2, sl3), (sg0, sg1, sg2, sg3))
        @pl.when(tid < SCAT_R)
        def _():
            r = NS * SCAT_Q + tid
            pltpu.sync_copy(dstr_i.at[r], ib0)
            pltpu.sync_copy(z.at[pl.ds(r * C, C)], zb0)
            pltpu.sync_copy(zb0, accum.at[ib0], add=True)
        plsc.subcore_barrier()
        wid = tid * NC + cid
        _gather_pipe(accum, src_i, a, wid * GATH_Q, GATH_Q,
                     (ib0, ib1, ib2, ib3), (zb0, zb1, zb2, zb3),
                     (sl0, sl1, sl2, sl3), (sg0, sg1, sg2, sg3),
                     (sw0, sw1, sw2, sw3))
        _gather_tail(accum, src_i, a, wid, ib0, zb0, sg0)

    return k(zs, dstr2, src2, zrows)


def _sc_final(e4f, dst2, zrows):
    @functools.partial(
        pl.kernel,
        out_type=jax.ShapeDtypeStruct((NC, N_PAD, 32), f32),
        mesh=plsc.VectorSubcoreMesh(**_MESH),
        compiler_params=_SC_PARAMS,
        scratch_types=[pltpu.VMEM_SHARED((N_PAD, 32), f32)] + _SC_SCRATCH,
    )
    def k(e, dst_i, zr, ffp, accum,
          ib0, ib1, ib2, ib3, zb0, zb1, zb2, zb3,
          sl0, sl1, sl2, sl3, sg0, sg1, sg2, sg3, sw0, sw1, sw2, sw3):
        cid = lax.axis_index("c")
        tid = lax.axis_index("s")
        pltpu.sync_copy(zr, accum.at[pl.ds(tid * 640, 640)])
        plsc.subcore_barrier()
        wid = tid * NC + cid
        _scatter_pipe(e, dst_i, accum, wid * GATH_Q, GATH_Q,
                      (ib0, ib1, ib2, ib3), (zb0, zb1, zb2, zb3),
                      (sl0, sl1, sl2, sl3), (sg0, sg1, sg2, sg3))
        _scatter_tail(e, dst_i, accum, wid, ib0, zb0)
        plsc.subcore_barrier()
        pltpu.sync_copy(accum.at[pl.ds(tid * 640, 640)],
                        ffp.at[cid, pl.ds(tid * 640, 640)])

    return k(e4f, dst2, zrows)


def kernel(n_feat, e_feat, edge_index, W_init, W_eupd, W_last):
    ei2 = edge_index.reshape(2 * ER, 128)
    src2, dst2, dstr2 = _tc_idx_prep(ei2)

    w4p = jnp.kron(jnp.eye(4, dtype=f32), W_eupd)[jnp.arange(128) ^ 32]
    we4 = jnp.kron(jnp.eye(4, dtype=f32), W_init[NODE_DIM:])
    zrows = jnp.zeros((640, 32), f32)

    p_tbl, r = _tc_node_pre(n_feat, W_init[:NODE_DIM], W_last[:NODE_DIM])
    q4 = _tc_q(e_feat.reshape(E4, 64), we4)
    g4 = _sc_gather(p_tbl, src2).reshape(E4, 128)
    e04, zs4 = _tc_step0(g4, q4, w4p)
    for _ in range(STEPS - 1):
        a4 = _sc_step(zs4.reshape(E, 32), dstr2, src2, zrows).reshape(E4, 128)
        zs4 = _tc_step_mid(e04, a4, zs4, w4p)
    a4 = _sc_step(zs4.reshape(E, 32), dstr2, src2, zrows).reshape(E4, 128)
    e4f = _tc_step_last(e04, a4, zs4)
    ffp = _sc_final(e4f.reshape(E, 32), dst2, zrows)
    out = _tc_final(r, ffp, W_last[NODE_DIM:])
    return out
